# baseline (speedup 1.0000x reference)
import functools

import jax
import jax.numpy as jnp
import numpy as np
from jax import lax
from jax.experimental import pallas as pl
from jax.experimental.pallas import tpu as pltpu
from jax.experimental.pallas import tpu_sc as plsc

D = 32
NC = 2
NS = 16
NW = NC * NS
AW = 128
BT = 8
BC = 4
SUB = 128
SCALE = np.float32(np.sqrt(np.float32(32.0)))


@functools.lru_cache(maxsize=None)
def _make_detile(V: int):
  IB = V // 128
  mesh = plsc.VectorSubcoreMesh(core_axis_name="c", subcore_axis_name="s")
  PW = -(-IB // NW)

  @functools.partial(
      pl.kernel,
      out_type=jax.ShapeDtypeStruct((V, D), jnp.float32),
      mesh=mesh,
      scratch_types=[
          [pltpu.VMEM((D // 8, 8, AW), jnp.float32) for _ in range(2)],
          [pltpu.VMEM((AW, D + 1), jnp.float32) for _ in range(2)],
          [pltpu.SemaphoreType.DMA for _ in range(2)],
          [pltpu.SemaphoreType.DMA for _ in range(2)],
      ],
      compiler_params=pltpu.CompilerParams(
          use_tc_tiling_on_sc=False, needs_layout_passes=False),
  )
  def detile_kernel(t4d_hbm, ltab_hbm, in_v, out_v, isem, osem):
    wid = lax.axis_index("s") * NC + lax.axis_index("c")
    lo = wid * PW
    hi = jnp.minimum(lo + PW, IB)
    iota = lax.iota(jnp.int32, 16)
    iotas = [iota + a0 * 16 for a0 in range(AW // 16)]
    jfulls = [jnp.full((16,), j, jnp.int32) for j in range(D)]

    def fire(ib, p):
      return pltpu.async_copy(
          t4d_hbm.at[pl.ds(0, D // 8), ib], in_v[p], isem[p])

    def step(ib, p, n):
      @pl.when(ib + 1 < hi)
      def _():
        fire(ib + 1, 1 - p)

      pltpu.make_async_copy(
          t4d_hbm.at[pl.ds(0, D // 8), 0], in_v[p], isem[p]).wait()

      @pl.when(n >= 2)
      def _():
        pltpu.make_async_copy(
            out_v[p].at[pl.ds(0, AW), pl.ds(0, D)],
            ltab_hbm.at[pl.ds(0, AW), pl.ds(0, D)], osem[p]).wait()

      for jb in range(D // 8):
        for j8 in range(8):
          jvec = jfulls[jb * 8 + j8]
          for a0 in range(AW // 16):
            v = in_v[p][jb, j8, pl.ds(a0 * 16, 16)] * SCALE
            plsc.store_scatter(out_v[p], [iotas[a0], jvec], v)

      pltpu.async_copy(
          out_v[p].at[pl.ds(0, AW), pl.ds(0, D)],
          ltab_hbm.at[pl.ds(ib * AW, AW), pl.ds(0, D)], osem[p])

    @pl.when(lo < hi)
    def _():
      fire(lo, 0)

      @pl.loop(0, PW, step=2)
      def ring(q):
        @pl.when(lo + q < hi)
        def _():
          step(lo + q, 0, q)

        @pl.when(lo + q + 1 < hi)
        def _():
          step(lo + q + 1, 1, q + 1)

      for p in range(2):
        pltpu.make_async_copy(
            out_v[p].at[pl.ds(0, AW), pl.ds(0, D)],
            ltab_hbm.at[pl.ds(0, AW), pl.ds(0, D)], osem[p]).wait()

  return detile_kernel


@functools.lru_cache(maxsize=None)
def _make_kernel(A: int, B: int):
  assert A == NW * AW and B % BC == 0 and BT % BC == 0
  G = B // BC
  CT = BC * AW
  BB, AB = B // BT, A // AW
  assert G % 2 == 0

  mesh = plsc.VectorSubcoreMesh(core_axis_name="c", subcore_axis_name="s")

  @functools.partial(
      pl.kernel,
      out_type=jax.ShapeDtypeStruct((B, D // 8, AB, 8, AW), jnp.float32),
      mesh=mesh,
      scratch_types=[
          [pltpu.VMEM((BC, AW), jnp.int32) for _ in range(2)],
          [pltpu.VMEM((CT, D), jnp.float32) for _ in range(2)],
          [pltpu.VMEM((BC, D // 8, 8, AW + 1), jnp.float32) for _ in range(2)],
          [pltpu.SemaphoreType.DMA for _ in range(2)],
          [pltpu.SemaphoreType.DMA for _ in range(2)],
      ],
      compiler_params=pltpu.CompilerParams(
          use_tc_tiling_on_sc=False, needs_layout_passes=False),
  )
  def emb_kernel(tokens_hbm, table_hbm, out_hbm, idx_v, rows_v, tr_v,
                 gsem, osem):
    wid = lax.axis_index("s") * NC + lax.axis_index("c")
    iota = lax.iota(jnp.int32, 16)
    jbvecs = [(iota + h * 16) // 8 for h in range(D // 16)]
    j8vecs = [(iota + h * 16) % 8 for h in range(D // 16)]

    def fire(g, p):
      bb = g // (BT // BC)
      bs = (g % (BT // BC)) * BC
      pltpu.sync_copy(
          tokens_hbm.at[bb, wid, pl.ds(bs, BC), pl.ds(0, AW)], idx_v[p])
      for s in range(BC):
        pltpu.async_copy(
            table_hbm.at[idx_v[p].at[s]],
            rows_v[p].at[pl.ds(s * AW, AW)],
            gsem[p],
        )

    def wait_gathers(p):
      for s in range(BC):
        pltpu.make_async_copy(
            table_hbm.at[idx_v[p].at[s]],
            rows_v[p].at[pl.ds(s * AW, AW)],
            gsem[p],
        ).wait()

    def out_slice(g):
      return out_hbm.at[pl.ds(g * BC, BC), pl.ds(0, D // 8), wid,
                        pl.ds(0, 8), pl.ds(0, AW)]

    def tr_slice(p):
      return tr_v[p].at[pl.ds(0, BC), pl.ds(0, D // 8), pl.ds(0, 8),
                        pl.ds(0, AW)]

    def step(g, p):
      @pl.when(g + 1 < G)
      def _():
        fire(g + 1, 1 - p)

      wait_gathers(p)

      @pl.when(g >= 2)
      def _():
        pltpu.make_async_copy(tr_slice(p), out_slice(0), osem[p]).wait()

      for b in range(BC):
        @pl.loop(0, AW, unroll=8)
        def tr_loop(a):
          avec = jnp.full((16,), a, jnp.int32)
          for h in range(D // 16):
            v = rows_v[p][b * AW + a, pl.ds(h * 16, 16)]
            plsc.store_scatter(
                tr_v[p].at[b], [jbvecs[h], j8vecs[h], avec], v)

      pltpu.async_copy(tr_slice(p), out_slice(g), osem[p])

    fire(0, 0)

    @pl.loop(0, G, step=2)
    def ring(q):
      step(q, 0)
      step(q + 1, 1)

    for p in range(2):
      pltpu.make_async_copy(tr_slice(p), out_slice(0), osem[p]).wait()

  return emb_kernel


@jax.jit
def kernel(tokens, table):
  A, B = tokens.shape
  tokens4d = jnp.transpose(
      tokens.reshape(A // AW, AW, B // BT, BT), (2, 0, 3, 1))
  V = table.shape[0] + (-table.shape[0]) % 128
  tpad = jnp.pad(table, ((0, V - table.shape[0]), (0, 0)))
  t4d = jnp.transpose(tpad.reshape(V // 128, 128, D // 8, 8), (2, 0, 3, 1))
  table_lin = _make_detile(V)(t4d)
  out5d = _make_kernel(A, B)(tokens4d, table_lin)
  out = jnp.transpose(out5d, (2, 4, 0, 1, 3)).reshape(A, B, D)
  return out


# speedup vs baseline: 1.1189x; 1.1189x over previous
import functools

import jax
import jax.numpy as jnp
import numpy as np
from jax import lax
from jax.experimental import pallas as pl
from jax.experimental.pallas import tpu as pltpu
from jax.experimental.pallas import tpu_sc as plsc

D = 32
NC = 2
NS = 16
NW = NC * NS
AW = 128
BT = 8
BC = 4
SUB = 128
SCALE = np.float32(np.sqrt(np.float32(32.0)))


@functools.lru_cache(maxsize=None)
def _make_kernel(A: int, B: int):
  assert A == NW * AW and B % BC == 0 and BT % BC == 0
  G = B // BC
  CT = BC * AW
  BB, AB = B // BT, A // AW
  assert G % 2 == 0

  mesh = plsc.VectorSubcoreMesh(core_axis_name="c", subcore_axis_name="s")

  @functools.partial(
      pl.kernel,
      out_type=jax.ShapeDtypeStruct((B, D // 8, AB, 8, AW), jnp.float32),
      mesh=mesh,
      scratch_types=[
          [pltpu.VMEM((BC, AW), jnp.int32) for _ in range(2)],
          [pltpu.VMEM((CT, D), jnp.float32) for _ in range(2)],
          [pltpu.VMEM((BC, D // 8, 8, AW + 1), jnp.float32) for _ in range(2)],
          [pltpu.SemaphoreType.DMA for _ in range(2)],
          [pltpu.SemaphoreType.DMA for _ in range(2)],
      ],
      compiler_params=pltpu.CompilerParams(
          use_tc_tiling_on_sc=False, needs_layout_passes=False),
  )
  def emb_kernel(tokens_hbm, table_hbm, out_hbm, idx_v, rows_v, tr_v,
                 gsem, osem):
    wid = lax.axis_index("s") * NC + lax.axis_index("c")
    iota = lax.iota(jnp.int32, 16)
    jbvecs = [(iota + h * 16) // 8 for h in range(D // 16)]
    j8vecs = [(iota + h * 16) % 8 for h in range(D // 16)]

    def fire(g, p):
      bb = g // (BT // BC)
      bs = (g % (BT // BC)) * BC
      pltpu.sync_copy(
          tokens_hbm.at[bb, wid, pl.ds(bs, BC), pl.ds(0, AW)], idx_v[p])
      for s in range(BC):
        pltpu.async_copy(
            table_hbm.at[idx_v[p].at[s]],
            rows_v[p].at[pl.ds(s * AW, AW)],
            gsem[p],
        )

    def wait_gathers(p):
      for s in range(BC):
        pltpu.make_async_copy(
            table_hbm.at[idx_v[p].at[s]],
            rows_v[p].at[pl.ds(s * AW, AW)],
            gsem[p],
        ).wait()

    def out_slice(g):
      return out_hbm.at[pl.ds(g * BC, BC), pl.ds(0, D // 8), wid,
                        pl.ds(0, 8), pl.ds(0, AW)]

    def tr_slice(p):
      return tr_v[p].at[pl.ds(0, BC), pl.ds(0, D // 8), pl.ds(0, 8),
                        pl.ds(0, AW)]

    def step(g, p):
      @pl.when(g + 1 < G)
      def _():
        fire(g + 1, 1 - p)

      wait_gathers(p)

      @pl.when(g >= 2)
      def _():
        pltpu.make_async_copy(tr_slice(p), out_slice(0), osem[p]).wait()

      for b in range(BC):
        @pl.loop(0, AW, unroll=8)
        def tr_loop(a):
          avec = jnp.full((16,), a, jnp.int32)
          for h in range(D // 16):
            v = rows_v[p][b * AW + a, pl.ds(h * 16, 16)] * SCALE
            plsc.store_scatter(
                tr_v[p].at[b], [jbvecs[h], j8vecs[h], avec], v)

      pltpu.async_copy(tr_slice(p), out_slice(g), osem[p])

    fire(0, 0)

    @pl.loop(0, G, step=2)
    def ring(q):
      step(q, 0)
      step(q + 1, 1)

    for p in range(2):
      pltpu.make_async_copy(tr_slice(p), out_slice(0), osem[p]).wait()

  return emb_kernel


@jax.jit
def kernel(tokens, table):
  A, B = tokens.shape
  tokens4d = jnp.transpose(
      tokens.reshape(A // AW, AW, B // BT, BT), (2, 0, 3, 1))
  out5d = _make_kernel(A, B)(tokens4d, table)
  out = jnp.transpose(out5d, (2, 4, 0, 1, 3)).reshape(A, B, D)
  return out
